# Initial kernel scaffold; baseline (speedup 1.0000x reference)
import jax
import jax.numpy as jnp
from jax.experimental import pallas as pl


def kernel(dense_x, sparse_x, tables, W0, b0, W1, b1, W2, b2, T0, c0, T1, c1, T2, c2):
    raise NotImplementedError("write your pallas kernel here")


# speedup vs baseline: 7.6922x; 7.6922x over previous
import functools

import jax
import jax.numpy as jnp
import numpy as np
from jax import lax
from jax.experimental import pallas as pl
from jax.experimental.pallas import tpu as pltpu
from jax.experimental.pallas import tpu_sc as plsc

_V = 100000
_NS = 26
_E = 32

_NC = 2
_NSUB = 16
_NW = _NC * _NSUB


def _sc_gather(tab_flat, idx4, rows, ch, nch, ipc):
    mesh = plsc.VectorSubcoreMesh(
        core_axis_name="c", subcore_axis_name="s",
        num_cores=_NC, num_subcores=_NSUB)
    rpw = rows // _NW

    @functools.partial(
        pl.kernel,
        out_type=jax.ShapeDtypeStruct((rows, _E), jnp.float32),
        mesh=mesh,
        compiler_params=pltpu.CompilerParams(use_tc_tiling_on_sc=False),
        scratch_types=[
            pltpu.VMEM((nch, ipc, 128), jnp.int32),
            pltpu.VMEM((ch, _E), jnp.float32),
            pltpu.SemaphoreType.DMA,
        ],
    )
    def gather_kernel(tab_hbm, idx_hbm, out_hbm, idx_v, buf, sem):
        wid = lax.axis_index("s") * _NC + lax.axis_index("c")
        pltpu.sync_copy(idx_hbm.at[wid], idx_v)
        base = wid * rpw

        @pl.loop(0, nch)
        def _chunk(c):
            handles = []
            for r in range(ipc):
                handles.append(pltpu.async_copy(
                    tab_hbm.at[idx_v.at[c, r]],
                    buf.at[pl.ds(r * 128, 128)], sem))
            for h in handles:
                h.wait()
            pltpu.sync_copy(buf, out_hbm.at[pl.ds(base + c * ch, ch)])

    return gather_kernel(tab_flat, idx4)


def _tc_dense(dense_x, emb2, W0, b0, W1, b1, W2, b2, Mt, T0d, c0, T1, c1, T2, c2):
    B = dense_x.shape[0]
    BLK = 1024
    grid = (B // BLK,)

    def body(dx_ref, emb_ref, W0r, b0r, W1r, b1r, W2r, b2r, Mtr, T0dr,
             c0r, T1r, c1r, T2r, c2r, out_ref):
        dx = dx_ref[...]
        h = jnp.maximum(dx @ W0r[...] + b0r[...], 0.0)
        h = jnp.maximum(h @ W1r[...] + b1r[...], 0.0)
        d = h @ W2r[...] + b2r[...]
        x = jnp.concatenate([d, emb_ref[...]], axis=1)
        x3 = x.reshape(BLK, 27, _E)
        z = lax.dot_general(x3, x3, (((2,), (2,)), ((0,), (0,))),
                            preferred_element_type=jnp.float32)
        zf = z.reshape(BLK, 729)
        t = jnp.maximum(zf @ Mtr[...] + d @ T0dr[...] + c0r[...], 0.0)
        t = jnp.maximum(t @ T1r[...] + c1r[...], 0.0)
        out_ref[...] = t @ T2r[...] + c2r[...]

    full = lambda a: pl.BlockSpec(a.shape, lambda i: (0,) * a.ndim)
    return pl.pallas_call(
        body,
        grid=grid,
        in_specs=[
            pl.BlockSpec((BLK, dense_x.shape[1]), lambda i: (i, 0)),
            pl.BlockSpec((BLK, emb2.shape[1]), lambda i: (i, 0)),
            full(W0), full(b0), full(W1), full(b1), full(W2), full(b2),
            full(Mt), full(T0d), full(c0), full(T1), full(c1), full(T2),
            full(c2),
        ],
        out_specs=pl.BlockSpec((BLK, 1), lambda i: (i, 0)),
        out_shape=jax.ShapeDtypeStruct((B, 1), jnp.float32),
    )(dense_x, emb2, W0, b0, W1, b1, W2, b2, Mt, T0d, c0, T1, c1, T2, c2)


def kernel(dense_x, sparse_x, tables, W0, b0, W1, b1, W2, b2, T0, c0, T1, c1, T2, c2):
    B = dense_x.shape[0]
    rows = B * _NS
    rpw = rows // _NW
    ch = 1024
    nch = rpw // ch
    ipc = ch // 128
    assert rows == _NW * nch * ipc * 128

    idx = jnp.mod(sparse_x, _V) + (jnp.arange(_NS, dtype=jnp.int32) * _V)[None, :]
    idx4 = idx.reshape(_NW, nch, ipc, 128)
    tab_flat = tables.reshape(_NS * _V, _E)

    emb_flat = _sc_gather(tab_flat, idx4, rows, ch, nch, ipc)
    emb2 = emb_flat.reshape(B, _NS * _E)

    iu0, iu1 = np.triu_indices(27, k=1)
    rows_m = jnp.asarray(iu0 * 27 + iu1, dtype=jnp.int32)
    Mt = jnp.zeros((729, 512), jnp.float32).at[rows_m].set(T0[:351])
    T0d = T0[351:]

    out2 = _tc_dense(dense_x, emb2, W0, b0.reshape(1, -1), W1, b1.reshape(1, -1),
                     W2, b2.reshape(1, -1), Mt, T0d, c0.reshape(1, -1),
                     T1, c1.reshape(1, -1), T2, c2.reshape(1, 1))
    return out2.reshape(B)
